# baseline (speedup 1.0000x reference)
import functools
import numpy as np
import jax
import jax.numpy as jnp
from jax import lax
from jax.experimental import pallas as pl
from jax.experimental.pallas import tpu as pltpu
from jax.experimental.pallas import tpu_sc as plsc

_N = 10000
_E = 160000
_EPAD = 163840
_NB_SCALE = float(np.sqrt(8.0) / 1.12)
_VALS = [(k + 1) * 5.0 / 9.0 for k in range(8)]
_INV_STEP = 9.0 / 5.0

_j = np.arange(384)
_T = np.zeros((8, 384), np.float32); _T[_j % 8, _j] = 1.0
_R = np.zeros((384, 48), np.float32); _R[_j, _j // 8] = 1.0
_q = np.arange(208); _o = _q // 13; _m = _q % 13
_Lm = (_m > 0).astype(np.int64) + (_m > 3).astype(np.int64)
_S = np.zeros((48, 208), np.float32); _S[_Lm * 16 + _o, _q] = 1.0
_U = np.zeros((13, 208), np.float32); _U[_m, _q] = 1.0


def _silu(x):
    return x * (1.0 / (1.0 + jnp.exp(-x)))


def _node_mlp_body(a_ref, emb_ref, w1_ref, b1_ref, w2_ref, b2_ref, out_ref):
    a = a_ref[...]
    oh = (a == lax.broadcasted_iota(jnp.int32, (a.shape[0], 100), 1))
    e = jnp.dot(oh.astype(jnp.float32), emb_ref[...],
                preferred_element_type=jnp.float32)
    h = _silu(jnp.dot(e, w1_ref[...], preferred_element_type=jnp.float32)
              + b1_ref[...])
    out_ref[...] = (jnp.dot(h, w2_ref[...], preferred_element_type=jnp.float32)
                    + b2_ref[...])


def _node_mlp(A2, emb, w1, b1, w2, b2):
    B = 2000
    z = lambda i: (0, 0)
    return pl.pallas_call(
        _node_mlp_body,
        grid=(_N // B,),
        in_specs=[
            pl.BlockSpec((B, 1), lambda i: (i, 0)),
            pl.BlockSpec((100, 16), z),
            pl.BlockSpec((16, 64), z),
            pl.BlockSpec((1, 64), z),
            pl.BlockSpec((64, 8), z),
            pl.BlockSpec((1, 8), z),
        ],
        out_specs=pl.BlockSpec((B, 8), lambda i: (i, 0)),
        out_shape=jax.ShapeDtypeStruct((_N, 8), jnp.float32),
    )(A2, emb, w1, b1, w2, b2)


_CH = 1024
_NCH = 5


def _sc_gather_body(pos_hbm, ai_hbm, src_hbm, dst_hbm,
                    xs_out, ps_out, pd_out,
                    sidx, didx, psb, pdb, xsb, sem):
    wid = lax.axis_index("c") * 16 + lax.axis_index("s")
    for ch in range(_NCH):
        base = wid * (_NCH * _CH) + ch * _CH
        brow = wid * (_NCH * 8) + ch * 8
        pltpu.sync_copy(src_hbm.at[pl.ds(brow, 8)], sidx)
        pltpu.sync_copy(dst_hbm.at[pl.ds(brow, 8)], didx)
        descs = []
        for j in range(8):
            sl = pl.ds(j * 128, 128)
            si = sidx.at[j]
            di = didx.at[j]
            descs.append(pltpu.async_copy(pos_hbm.at[si], psb.at[sl], sem))
            descs.append(pltpu.async_copy(pos_hbm.at[di], pdb.at[sl], sem))
            descs.append(pltpu.async_copy(ai_hbm.at[si], xsb.at[sl], sem))
        for d in descs:
            d.wait()
        esl = pl.ds(base, _CH)
        pltpu.sync_copy(xsb, xs_out.at[esl])
        pltpu.sync_copy(psb, ps_out.at[esl])
        pltpu.sync_copy(pdb, pd_out.at[esl])


def _edge_dense_body(ps_ref, pd_ref, xs_ref, w1_ref, b1_ref, w2_ref, b2_ref,
                     w3_ref, b3_ref, t_ref, r_ref, s_ref, u_ref, out_ref):
    f32 = jnp.float32
    B = xs_ref.shape[0]
    d = pd_ref[...][:, 0:3] - ps_ref[...][:, 0:3]
    r2 = jnp.sum(d * d, axis=1, keepdims=True) + 1e-12
    ln = jnp.sqrt(r2)
    rinv = 1.0 / ln
    u = d * rinv
    vals = ((lax.broadcasted_iota(jnp.int32, (1, 8), 1).astype(f32) + 1.0)
            * (5.0 / 9.0))
    t = (ln - vals) * _INV_STEP
    er = jnp.exp(-(t * t)) * _NB_SCALE
    mono = jnp.concatenate(
        [jnp.full((B, 1), 0.0625, f32), u * 0.0625,
         jnp.concatenate([u[:, 0:1] * u, u[:, 1:2] * u, u[:, 2:3] * u],
                         axis=1) * 0.0625], axis=1)
    h = _silu(jnp.dot(er, w1_ref[...], preferred_element_type=f32)
              + b1_ref[...])
    h = _silu(jnp.dot(h, w2_ref[...], preferred_element_type=f32)
              + b2_ref[...])
    w3o = jnp.dot(h, w3_ref[...], preferred_element_type=f32) + b3_ref[...]
    xt = jnp.dot(xs_ref[...], t_ref[...], preferred_element_type=f32)
    coeff = jnp.dot(w3o * xt, r_ref[...], preferred_element_type=f32)
    res = (jnp.dot(coeff, s_ref[...], preferred_element_type=f32)
           * jnp.dot(mono, u_ref[...], preferred_element_type=f32))
    out_ref[0, :, 0:104] = res[:, 0:104]
    out_ref[1, :, 0:104] = res[:, 104:208]


def _edge_dense(ps, pd, xs, w1, b1, w2, b2, w3, b3):
    B = 2048
    z = lambda i: (0, 0)
    return pl.pallas_call(
        _edge_dense_body,
        grid=(_EPAD // B,),
        in_specs=[
            pl.BlockSpec((B, 8), lambda i: (i, 0)),
            pl.BlockSpec((B, 8), lambda i: (i, 0)),
            pl.BlockSpec((B, 8), lambda i: (i, 0)),
            pl.BlockSpec((8, 64), z),
            pl.BlockSpec((1, 64), z),
            pl.BlockSpec((64, 64), z),
            pl.BlockSpec((1, 64), z),
            pl.BlockSpec((64, 384), z),
            pl.BlockSpec((1, 384), z),
            pl.BlockSpec((8, 384), z),
            pl.BlockSpec((384, 48), z),
            pl.BlockSpec((48, 208), z),
            pl.BlockSpec((13, 208), z),
        ],
        out_specs=pl.BlockSpec((2, B, 128), lambda i: (0, i, 0)),
        out_shape=jax.ShapeDtypeStruct((2, _EPAD, 128), jnp.float32),
    )(ps, pd, xs, w1, b1, w2, b2, w3, b3,
      jnp.asarray(_T), jnp.asarray(_R), jnp.asarray(_S), jnp.asarray(_U))


_NACC = 10112


def _sc_scatter_body(feat_hbm, dst_hbm, zeros_hbm, out_hbm,
                     didx, fb0, fb1, acc, sem0, sem1):
    cid = lax.axis_index("c")
    tid = lax.axis_index("s")
    rsl = pl.ds(pl.multiple_of(tid * 632, 8), 632)
    pltpu.sync_copy(zeros_hbm, acc.at[rsl])
    pltpu.sync_copy(dst_hbm.at[pl.ds(pl.multiple_of(tid * 80, 8), 80)], didx)
    plsc.subcore_barrier()
    fbs = [fb0, fb1]
    sems = [sem0, sem1]

    def feat_src(h):
        e0 = pl.multiple_of(tid * 10240 + h * 128, 8)
        return feat_hbm.at[cid, pl.ds(e0, 128), :]

    pend = pltpu.async_copy(feat_src(0), fb0, sem0)
    for h in range(80):
        nxt = (pltpu.async_copy(feat_src(h + 1), fbs[(h + 1) % 2],
                                sems[(h + 1) % 2]) if h < 79 else None)
        pend.wait()
        pltpu.sync_copy(fbs[h % 2], acc.at[didx.at[h]], add=True)
        pend = nxt
    plsc.subcore_barrier()
    pltpu.sync_copy(acc.at[rsl], out_hbm.at[cid, rsl])


_SC_CACHE = []


def _get_sc_kernels():
    if not _SC_CACHE:
        mesh = plsc.VectorSubcoreMesh(core_axis_name="c", subcore_axis_name="s",
                                      num_cores=2, num_subcores=16)
        gather = functools.partial(
            pl.kernel,
            out_type=[jax.ShapeDtypeStruct((_EPAD, 8), jnp.float32),
                      jax.ShapeDtypeStruct((_EPAD, 8), jnp.float32),
                      jax.ShapeDtypeStruct((_EPAD, 8), jnp.float32)],
            mesh=mesh,
            scratch_types=[pltpu.VMEM((8, 128), jnp.int32),
                           pltpu.VMEM((8, 128), jnp.int32),
                           pltpu.VMEM((_CH, 8), jnp.float32),
                           pltpu.VMEM((_CH, 8), jnp.float32),
                           pltpu.VMEM((_CH, 8), jnp.float32),
                           pltpu.SemaphoreType.DMA],
            compiler_params=pltpu.CompilerParams(use_tc_tiling_on_sc=False),
        )(_sc_gather_body)
        scatter = functools.partial(
            pl.kernel,
            out_type=jax.ShapeDtypeStruct((2, _NACC, 128), jnp.float32),
            mesh=mesh,
            scratch_types=[pltpu.VMEM((80, 128), jnp.int32),
                           pltpu.VMEM((128, 128), jnp.float32),
                           pltpu.VMEM((128, 128), jnp.float32),
                           pltpu.VMEM_SHARED((_NACC, 128), jnp.float32),
                           pltpu.SemaphoreType.DMA,
                           pltpu.SemaphoreType.DMA],
        )(_sc_scatter_body)
        _SC_CACHE.append((gather, scatter))
    return _SC_CACHE[0]


def kernel(pos, edge_shifts, cell, emb_table, mlp_w1, mlp_b1, mlp_w2, mlp_b2,
           fc_w1, fc_b1, fc_w2, fc_b2, fc_w3, fc_b3,
           A, batch, edge_src, edge_dst):
    Ai = _node_mlp(A.astype(jnp.int32).reshape(_N, 1), emb_table,
                   mlp_w1, mlp_b1.reshape(1, 64), mlp_w2, mlp_b2.reshape(1, 8))
    ai_ext = jnp.concatenate([Ai, jnp.zeros((1, 8), jnp.float32)], axis=0)
    pos_ext = jnp.zeros((_N + 1, 8), jnp.float32).at[:_N, :3].set(pos)
    npad = _EPAD - _E
    src2 = jnp.concatenate([edge_src.astype(jnp.int32),
                            jnp.full((npad,), _N, jnp.int32)]).reshape(-1, 128)
    dst2 = jnp.concatenate([edge_dst.astype(jnp.int32),
                            jnp.zeros((npad,), jnp.int32)]).reshape(-1, 128)
    sc_gather, sc_scatter = _get_sc_kernels()
    xs, ps, pd = sc_gather(pos_ext, ai_ext, src2, dst2)
    feat = _edge_dense(ps, pd, xs, fc_w1, fc_b1.reshape(1, 64),
                       fc_w2, fc_b2.reshape(1, 64),
                       fc_w3, fc_b3.reshape(1, 384))
    acc2 = sc_scatter(feat, dst2, jnp.zeros((632, 128), jnp.float32))
    acc = jnp.concatenate([acc2[0, :_N, :104], acc2[1, :_N, :104]], axis=1)
    acc3 = acc.reshape(_N, 16, 13)
    blk0 = acc3[:, :, 0]
    blk1 = acc3[:, :, 1:4].reshape(_N, 48)
    blk2 = acc3[:, :, 4:13].reshape(_N, 144)
    return jnp.concatenate(
        [blk0, jnp.zeros((_N, 16), jnp.float32),
         blk1, jnp.zeros((_N, 48), jnp.float32),
         blk2, jnp.zeros((_N, 144), jnp.float32)], axis=1)


# speedup vs baseline: 1.2499x; 1.2499x over previous
import functools
import numpy as np
import jax
import jax.numpy as jnp
from jax import lax
from jax.experimental import pallas as pl
from jax.experimental.pallas import tpu as pltpu
from jax.experimental.pallas import tpu_sc as plsc

_N = 10000
_E = 160000
_EPAD = 163840
_NB_SCALE = float(np.sqrt(8.0) / 1.12)
_VALS = [(k + 1) * 5.0 / 9.0 for k in range(8)]
_INV_STEP = 9.0 / 5.0

_j = np.arange(384)
_T = np.zeros((8, 384), np.float32); _T[_j % 8, _j] = 1.0
_R = np.zeros((384, 48), np.float32); _R[_j, _j // 8] = 1.0
_q = np.arange(208); _o = _q // 13; _m = _q % 13
_Lm = (_m > 0).astype(np.int64) + (_m > 3).astype(np.int64)
_S = np.zeros((48, 208), np.float32); _S[_Lm * 16 + _o, _q] = 1.0
_U = np.zeros((13, 208), np.float32); _U[_m, _q] = 1.0


def _silu(x):
    return x * (1.0 / (1.0 + jnp.exp(-x)))


def _node_mlp_body(a_ref, emb_ref, w1_ref, b1_ref, w2_ref, b2_ref, out_ref):
    a = a_ref[...]
    oh = (a == lax.broadcasted_iota(jnp.int32, (a.shape[0], 100), 1))
    e = jnp.dot(oh.astype(jnp.float32), emb_ref[...],
                preferred_element_type=jnp.float32)
    h = _silu(jnp.dot(e, w1_ref[...], preferred_element_type=jnp.float32)
              + b1_ref[...])
    out_ref[...] = (jnp.dot(h, w2_ref[...], preferred_element_type=jnp.float32)
                    + b2_ref[...])


def _node_mlp(A2, emb, w1, b1, w2, b2):
    B = 2000
    z = lambda i: (0, 0)
    return pl.pallas_call(
        _node_mlp_body,
        grid=(_N // B,),
        in_specs=[
            pl.BlockSpec((B, 1), lambda i: (i, 0)),
            pl.BlockSpec((100, 16), z),
            pl.BlockSpec((16, 64), z),
            pl.BlockSpec((1, 64), z),
            pl.BlockSpec((64, 8), z),
            pl.BlockSpec((1, 8), z),
        ],
        out_specs=pl.BlockSpec((B, 8), lambda i: (i, 0)),
        out_shape=jax.ShapeDtypeStruct((_N, 8), jnp.float32),
    )(A2, emb, w1, b1, w2, b2)


_CH = 1024
_NCH = 5


def _sc_gather_body(px_hbm, py_hbm, pz_hbm, ai_hbm, src_hbm, dst_hbm,
                    xs_out, p_out,
                    sidx, didx, psx, psy, psz, pdx, pdy, pdz, xsb, sem):
    wid = lax.axis_index("c") * 16 + lax.axis_index("s")
    for ch in range(_NCH):
        base = wid * (_NCH * _CH) + ch * _CH
        brow = wid * (_NCH * 8) + ch * 8
        pltpu.sync_copy(src_hbm.at[pl.ds(brow, 8)], sidx)
        pltpu.sync_copy(dst_hbm.at[pl.ds(brow, 8)], didx)
        descs = []
        for j in range(8):
            sl = pl.ds(j * 128, 128)
            si = sidx.at[j]
            di = didx.at[j]
            descs.append(pltpu.async_copy(px_hbm.at[si], psx.at[sl], sem))
            descs.append(pltpu.async_copy(py_hbm.at[si], psy.at[sl], sem))
            descs.append(pltpu.async_copy(pz_hbm.at[si], psz.at[sl], sem))
            descs.append(pltpu.async_copy(px_hbm.at[di], pdx.at[sl], sem))
            descs.append(pltpu.async_copy(py_hbm.at[di], pdy.at[sl], sem))
            descs.append(pltpu.async_copy(pz_hbm.at[di], pdz.at[sl], sem))
            descs.append(pltpu.async_copy(ai_hbm.at[si], xsb.at[sl], sem))
        for d in descs:
            d.wait()
        esl = pl.ds(base, _CH)
        pltpu.sync_copy(xsb, xs_out.at[esl])
        pltpu.sync_copy(psx, p_out.at[0, esl])
        pltpu.sync_copy(psy, p_out.at[1, esl])
        pltpu.sync_copy(psz, p_out.at[2, esl])
        pltpu.sync_copy(pdx, p_out.at[3, esl])
        pltpu.sync_copy(pdy, p_out.at[4, esl])
        pltpu.sync_copy(pdz, p_out.at[5, esl])


def _tdot(a, b, precision=None):
    return lax.dot_general(a, b, (((0,), (0,)), ((), ())),
                           preferred_element_type=jnp.float32,
                           precision=precision)


def _edge_dense_body(p_ref, xs_ref, w1_ref, b1_ref, w2_ref, b2_ref,
                     w3_ref, b3_ref, t_ref, r_ref, s_ref, u_ref, out_ref):
    f32 = jnp.float32
    B = xs_ref.shape[0]
    p = p_ref[...]
    d = p[3:6, :] - p[0:3, :]
    r2 = jnp.sum(d * d, axis=0, keepdims=True) + 1e-12
    ln = jnp.sqrt(r2)
    rinv = 1.0 / ln
    u = d * rinv
    vals = ((lax.broadcasted_iota(jnp.int32, (8, 1), 0).astype(f32) + 1.0)
            * (5.0 / 9.0))
    t = (ln - vals) * _INV_STEP
    er = jnp.exp(-(t * t)) * _NB_SCALE
    mono = jnp.concatenate(
        [jnp.full((1, B), 0.0625, f32), u * 0.0625,
         jnp.concatenate([u[0:1] * u, u[1:2] * u, u[2:3] * u],
                         axis=0) * 0.0625], axis=0)
    h = _silu(_tdot(er, w1_ref[...]) + b1_ref[...])
    h = _silu(jnp.dot(h, w2_ref[...], preferred_element_type=f32)
              + b2_ref[...])
    w3o = jnp.dot(h, w3_ref[...], preferred_element_type=f32) + b3_ref[...]
    xt = jnp.dot(xs_ref[...], t_ref[...], preferred_element_type=f32)
    coeff = jnp.dot(w3o * xt, r_ref[...], preferred_element_type=f32)
    res = (jnp.dot(coeff, s_ref[...], preferred_element_type=f32)
           * _tdot(mono, u_ref[...]))
    out_ref[0, :, 0:104] = res[:, 0:104]
    out_ref[1, :, 0:104] = res[:, 104:208]


def _edge_dense(p, xs, w1, b1, w2, b2, w3, b3):
    B = 2048
    z = lambda i: (0, 0)
    return pl.pallas_call(
        _edge_dense_body,
        grid=(_EPAD // B,),
        in_specs=[
            pl.BlockSpec((6, B), lambda i: (0, i)),
            pl.BlockSpec((B, 8), lambda i: (i, 0)),
            pl.BlockSpec((8, 64), z),
            pl.BlockSpec((1, 64), z),
            pl.BlockSpec((64, 64), z),
            pl.BlockSpec((1, 64), z),
            pl.BlockSpec((64, 384), z),
            pl.BlockSpec((1, 384), z),
            pl.BlockSpec((8, 384), z),
            pl.BlockSpec((384, 48), z),
            pl.BlockSpec((48, 208), z),
            pl.BlockSpec((13, 208), z),
        ],
        out_specs=pl.BlockSpec((2, B, 128), lambda i: (0, i, 0)),
        out_shape=jax.ShapeDtypeStruct((2, _EPAD, 128), jnp.float32),
    )(p, xs, w1, b1, w2, b2, w3, b3,
      jnp.asarray(_T), jnp.asarray(_R), jnp.asarray(_S), jnp.asarray(_U))


_NACC = 10112


def _sc_scatter_body(feat_hbm, dst_hbm, zeros_hbm, out_hbm,
                     didx, fb0, fb1, acc, semr0, semr1, sems0, sems1):
    cid = lax.axis_index("c")
    tid = lax.axis_index("s")
    rsl = pl.ds(pl.multiple_of(tid * 632, 8), 632)
    pltpu.sync_copy(zeros_hbm, acc.at[rsl])
    pltpu.sync_copy(dst_hbm.at[pl.ds(pl.multiple_of(tid * 80, 8), 80)], didx)
    plsc.subcore_barrier()
    fbs = [fb0, fb1]
    semr = [semr0, semr1]
    sems = [sems0, sems1]

    def feat_src(h):
        e0 = pl.multiple_of(tid * 10240 + h * 128, 8)
        return feat_hbm.at[cid, pl.ds(e0, 128), :]

    pend_r = [pltpu.async_copy(feat_src(0), fb0, semr0), None]
    pend_s = [None, None]
    for h in range(80):
        p = h % 2
        q = (h + 1) % 2
        if h < 79:
            if pend_s[q] is not None:
                pend_s[q].wait()
            pend_r[q] = pltpu.async_copy(feat_src(h + 1), fbs[q], semr[q])
        pend_r[p].wait()
        pend_s[p] = pltpu.async_copy(fbs[p], acc.at[didx.at[h]], sems[p],
                                     add=True)
    pend_s[0].wait()
    pend_s[1].wait()
    plsc.subcore_barrier()
    pltpu.sync_copy(acc.at[rsl], out_hbm.at[cid, rsl])


_SC_CACHE = []


def _get_sc_kernels():
    if not _SC_CACHE:
        mesh = plsc.VectorSubcoreMesh(core_axis_name="c", subcore_axis_name="s",
                                      num_cores=2, num_subcores=16)
        gather = functools.partial(
            pl.kernel,
            out_type=[jax.ShapeDtypeStruct((_EPAD, 8), jnp.float32),
                      jax.ShapeDtypeStruct((6, _EPAD), jnp.float32)],
            mesh=mesh,
            scratch_types=[pltpu.VMEM((8, 128), jnp.int32),
                           pltpu.VMEM((8, 128), jnp.int32),
                           pltpu.VMEM((_CH,), jnp.float32),
                           pltpu.VMEM((_CH,), jnp.float32),
                           pltpu.VMEM((_CH,), jnp.float32),
                           pltpu.VMEM((_CH,), jnp.float32),
                           pltpu.VMEM((_CH,), jnp.float32),
                           pltpu.VMEM((_CH,), jnp.float32),
                           pltpu.VMEM((_CH, 8), jnp.float32),
                           pltpu.SemaphoreType.DMA],
            compiler_params=pltpu.CompilerParams(use_tc_tiling_on_sc=False),
        )(_sc_gather_body)
        scatter = functools.partial(
            pl.kernel,
            out_type=jax.ShapeDtypeStruct((2, _NACC, 128), jnp.float32),
            mesh=mesh,
            scratch_types=[pltpu.VMEM((80, 128), jnp.int32),
                           pltpu.VMEM((128, 128), jnp.float32),
                           pltpu.VMEM((128, 128), jnp.float32),
                           pltpu.VMEM_SHARED((_NACC, 128), jnp.float32),
                           pltpu.SemaphoreType.DMA,
                           pltpu.SemaphoreType.DMA,
                           pltpu.SemaphoreType.DMA,
                           pltpu.SemaphoreType.DMA],
        )(_sc_scatter_body)
        _SC_CACHE.append((gather, scatter))
    return _SC_CACHE[0]


def kernel(pos, edge_shifts, cell, emb_table, mlp_w1, mlp_b1, mlp_w2, mlp_b2,
           fc_w1, fc_b1, fc_w2, fc_b2, fc_w3, fc_b3,
           A, batch, edge_src, edge_dst):
    Ai = _node_mlp(A.astype(jnp.int32).reshape(_N, 1), emb_table,
                   mlp_w1, mlp_b1.reshape(1, 64), mlp_w2, mlp_b2.reshape(1, 8))
    ai_ext = jnp.concatenate([Ai, jnp.zeros((1, 8), jnp.float32)], axis=0)
    posp = jnp.concatenate([pos, jnp.zeros((1, 3), jnp.float32)], axis=0)
    px = posp[:, 0]
    py = posp[:, 1]
    pz = posp[:, 2]
    npad = _EPAD - _E
    src2 = jnp.concatenate([edge_src.astype(jnp.int32),
                            jnp.full((npad,), _N, jnp.int32)]).reshape(-1, 128)
    dst2 = jnp.concatenate([edge_dst.astype(jnp.int32),
                            jnp.zeros((npad,), jnp.int32)]).reshape(-1, 128)
    sc_gather, sc_scatter = _get_sc_kernels()
    xs, pgath = sc_gather(px, py, pz, ai_ext, src2, dst2)
    feat = _edge_dense(pgath, xs, fc_w1, fc_b1.reshape(1, 64),
                       fc_w2, fc_b2.reshape(1, 64),
                       fc_w3, fc_b3.reshape(1, 384))
    acc2 = sc_scatter(feat, dst2, jnp.zeros((632, 128), jnp.float32))
    acc = jnp.concatenate([acc2[0, :_N, :104], acc2[1, :_N, :104]], axis=1)
    acc3 = acc.reshape(_N, 16, 13)
    blk0 = acc3[:, :, 0]
    blk1 = acc3[:, :, 1:4].reshape(_N, 48)
    blk2 = acc3[:, :, 4:13].reshape(_N, 144)
    return jnp.concatenate(
        [blk0, jnp.zeros((_N, 16), jnp.float32),
         blk1, jnp.zeros((_N, 48), jnp.float32),
         blk2, jnp.zeros((_N, 144), jnp.float32)], axis=1)


# speedup vs baseline: 1.3797x; 1.1039x over previous
import functools
import numpy as np
import jax
import jax.numpy as jnp
from jax import lax
from jax.experimental import pallas as pl
from jax.experimental.pallas import tpu as pltpu
from jax.experimental.pallas import tpu_sc as plsc

_N = 10000
_E = 160000
_EPAD = 163840
_NB_SCALE = float(np.sqrt(8.0) / 1.12)
_VALS = [(k + 1) * 5.0 / 9.0 for k in range(8)]
_INV_STEP = 9.0 / 5.0

_j = np.arange(384)
_T = np.zeros((8, 384), np.float32); _T[_j % 8, _j] = 1.0
_R = np.zeros((384, 48), np.float32); _R[_j, _j // 8] = 1.0
_q = np.arange(208); _o = _q // 13; _m = _q % 13
_Lm = (_m > 0).astype(np.int64) + (_m > 3).astype(np.int64)
_S = np.zeros((48, 208), np.float32); _S[_Lm * 16 + _o, _q] = 1.0
_U = np.zeros((13, 208), np.float32); _U[_m, _q] = 1.0


def _silu(x):
    return x * (1.0 / (1.0 + jnp.exp(-x)))


def _node_mlp_body(a_ref, emb_ref, w1_ref, b1_ref, w2_ref, b2_ref, out_ref):
    a = a_ref[...]
    oh = (a == lax.broadcasted_iota(jnp.int32, (a.shape[0], 100), 1))
    e = jnp.dot(oh.astype(jnp.float32), emb_ref[...],
                preferred_element_type=jnp.float32)
    h = _silu(jnp.dot(e, w1_ref[...], preferred_element_type=jnp.float32)
              + b1_ref[...])
    out_ref[...] = (jnp.dot(h, w2_ref[...], preferred_element_type=jnp.float32)
                    + b2_ref[...])


def _node_mlp(A2, emb, w1, b1, w2, b2):
    B = 2000
    z = lambda i: (0, 0)
    return pl.pallas_call(
        _node_mlp_body,
        grid=(_N // B,),
        in_specs=[
            pl.BlockSpec((B, 1), lambda i: (i, 0)),
            pl.BlockSpec((100, 16), z),
            pl.BlockSpec((16, 64), z),
            pl.BlockSpec((1, 64), z),
            pl.BlockSpec((64, 8), z),
            pl.BlockSpec((1, 8), z),
        ],
        out_specs=pl.BlockSpec((B, 8), lambda i: (i, 0)),
        out_shape=jax.ShapeDtypeStruct((_N, 8), jnp.float32),
    )(A2, emb, w1, b1, w2, b2)


_CH = 1024


def _make_sc_gather_body(nch):
  def _sc_gather_body(px_hbm, py_hbm, pz_hbm, ai_hbm, src_hbm, dst_hbm,
                      xs_out, p_out,
                      sidx, didx, psx, psy, psz, pdx, pdy, pdz, xsb, sem):
    wid = lax.axis_index("c") * 16 + lax.axis_index("s")
    for ch in range(nch):
        base = wid * (nch * _CH) + ch * _CH
        brow = wid * (nch * 8) + ch * 8
        pltpu.sync_copy(src_hbm.at[pl.ds(brow, 8)], sidx)
        pltpu.sync_copy(dst_hbm.at[pl.ds(brow, 8)], didx)
        descs = []
        for j in range(8):
            sl = pl.ds(j * 128, 128)
            si = sidx.at[j]
            di = didx.at[j]
            descs.append(pltpu.async_copy(px_hbm.at[si], psx.at[sl], sem))
            descs.append(pltpu.async_copy(py_hbm.at[si], psy.at[sl], sem))
            descs.append(pltpu.async_copy(pz_hbm.at[si], psz.at[sl], sem))
            descs.append(pltpu.async_copy(px_hbm.at[di], pdx.at[sl], sem))
            descs.append(pltpu.async_copy(py_hbm.at[di], pdy.at[sl], sem))
            descs.append(pltpu.async_copy(pz_hbm.at[di], pdz.at[sl], sem))
            descs.append(pltpu.async_copy(ai_hbm.at[si], xsb.at[sl], sem))
        for d in descs:
            d.wait()
        esl = pl.ds(base, _CH)
        pltpu.sync_copy(xsb, xs_out.at[esl])
        pltpu.sync_copy(psx, p_out.at[0, esl])
        pltpu.sync_copy(psy, p_out.at[1, esl])
        pltpu.sync_copy(psz, p_out.at[2, esl])
        pltpu.sync_copy(pdx, p_out.at[3, esl])
        pltpu.sync_copy(pdy, p_out.at[4, esl])
        pltpu.sync_copy(pdz, p_out.at[5, esl])
  return _sc_gather_body


def _tdot(a, b, precision=None):
    return lax.dot_general(a, b, (((0,), (0,)), ((), ())),
                           preferred_element_type=jnp.float32,
                           precision=precision)


def _edge_dense_body(p_ref, xs_ref, w1_ref, b1_ref, w2_ref, b2_ref,
                     w3_ref, b3_ref, t_ref, r_ref, s_ref, u_ref, out_ref):
    f32 = jnp.float32
    B = xs_ref.shape[0]
    p = p_ref[...]
    d = p[3:6, :] - p[0:3, :]
    r2 = jnp.sum(d * d, axis=0, keepdims=True) + 1e-12
    ln = jnp.sqrt(r2)
    rinv = 1.0 / ln
    u = d * rinv
    vals = ((lax.broadcasted_iota(jnp.int32, (8, 1), 0).astype(f32) + 1.0)
            * (5.0 / 9.0))
    t = (ln - vals) * _INV_STEP
    er = jnp.exp(-(t * t)) * _NB_SCALE
    mono = jnp.concatenate(
        [jnp.full((1, B), 0.0625, f32), u * 0.0625,
         jnp.concatenate([u[0:1] * u, u[1:2] * u, u[2:3] * u],
                         axis=0) * 0.0625], axis=0)
    h = _silu(_tdot(er, w1_ref[...]) + b1_ref[...])
    h = _silu(jnp.dot(h, w2_ref[...], preferred_element_type=f32)
              + b2_ref[...])
    w3o = jnp.dot(h, w3_ref[...], preferred_element_type=f32) + b3_ref[...]
    xt = jnp.dot(xs_ref[...], t_ref[...], preferred_element_type=f32)
    coeff = jnp.dot(w3o * xt, r_ref[...], preferred_element_type=f32)
    res = (jnp.dot(coeff, s_ref[...], preferred_element_type=f32)
           * _tdot(mono, u_ref[...]))
    out_ref[0, :, 0:104] = res[:, 0:104]
    out_ref[1, :, 0:104] = res[:, 104:208]


def _edge_dense(p, xs, w1, b1, w2, b2, w3, b3):
    ne = xs.shape[0]
    B = 2048
    z = lambda i: (0, 0)
    return pl.pallas_call(
        _edge_dense_body,
        grid=(ne // B,),
        in_specs=[
            pl.BlockSpec((6, B), lambda i: (0, i)),
            pl.BlockSpec((B, 8), lambda i: (i, 0)),
            pl.BlockSpec((8, 64), z),
            pl.BlockSpec((1, 64), z),
            pl.BlockSpec((64, 64), z),
            pl.BlockSpec((1, 64), z),
            pl.BlockSpec((64, 384), z),
            pl.BlockSpec((1, 384), z),
            pl.BlockSpec((8, 384), z),
            pl.BlockSpec((384, 48), z),
            pl.BlockSpec((48, 208), z),
            pl.BlockSpec((13, 208), z),
        ],
        out_specs=pl.BlockSpec((2, B, 128), lambda i: (0, i, 0)),
        out_shape=jax.ShapeDtypeStruct((2, ne, 128), jnp.float32),
    )(p, xs, w1, b1, w2, b2, w3, b3,
      jnp.asarray(_T), jnp.asarray(_R), jnp.asarray(_S), jnp.asarray(_U))


_NACC = 10112


def _make_sc_scatter_body(nsub):
  nh = nsub * 16
  def _sc_scatter_body(feat_hbm, dst_hbm, zeros_hbm, out_hbm,
                       didx, fb0, fb1, acc, semr0, semr1, sems0, sems1):
    cid = lax.axis_index("c")
    tid = lax.axis_index("s")
    rsl = pl.ds(pl.multiple_of(tid * 632, 8), 632)
    pltpu.sync_copy(zeros_hbm, acc.at[rsl])
    pltpu.sync_copy(dst_hbm.at[pl.ds(pl.multiple_of(tid * nh, 8), nh)], didx)
    plsc.subcore_barrier()
    fbs = [fb0, fb1]
    semr = [semr0, semr1]
    sems = [sems0, sems1]

    def feat_src(h):
        e0 = pl.multiple_of((tid * nh + h) * 128, 8)
        return feat_hbm.at[cid, pl.ds(e0, 128), :]

    pend_r = [pltpu.async_copy(feat_src(0), fb0, semr0), None]
    pend_s = [None, None]
    for h in range(nh):
        p = h % 2
        q = (h + 1) % 2
        if h < nh - 1:
            if pend_s[q] is not None:
                pend_s[q].wait()
            pend_r[q] = pltpu.async_copy(feat_src(h + 1), fbs[q], semr[q])
        pend_r[p].wait()
        pend_s[p] = pltpu.async_copy(fbs[p], acc.at[didx.at[h]], sems[p],
                                     add=True)
    pend_s[0].wait()
    pend_s[1].wait()
    plsc.subcore_barrier()
    pltpu.sync_copy(acc.at[rsl], out_hbm.at[cid, rsl])
  return _sc_scatter_body


_SC_CACHE = []


def _get_sc_kernels():
    if not _SC_CACHE:
        mesh = plsc.VectorSubcoreMesh(core_axis_name="c", subcore_axis_name="s",
                                      num_cores=2, num_subcores=16)
        kernels = {}
        for nsub in (2, 3):
            ne = nsub * 32 * _CH
            kernels[("g", nsub)] = functools.partial(
                pl.kernel,
                out_type=[jax.ShapeDtypeStruct((ne, 8), jnp.float32),
                          jax.ShapeDtypeStruct((6, ne), jnp.float32)],
                mesh=mesh,
                scratch_types=[pltpu.VMEM((8, 128), jnp.int32),
                               pltpu.VMEM((8, 128), jnp.int32),
                               pltpu.VMEM((_CH,), jnp.float32),
                               pltpu.VMEM((_CH,), jnp.float32),
                               pltpu.VMEM((_CH,), jnp.float32),
                               pltpu.VMEM((_CH,), jnp.float32),
                               pltpu.VMEM((_CH,), jnp.float32),
                               pltpu.VMEM((_CH,), jnp.float32),
                               pltpu.VMEM((_CH, 8), jnp.float32),
                               pltpu.SemaphoreType.DMA],
                compiler_params=pltpu.CompilerParams(use_tc_tiling_on_sc=False),
            )(_make_sc_gather_body(nsub))
            kernels[("s", nsub)] = functools.partial(
                pl.kernel,
                out_type=jax.ShapeDtypeStruct((2, _NACC, 128), jnp.float32),
                mesh=mesh,
                scratch_types=[pltpu.VMEM((nsub * 16, 128), jnp.int32),
                               pltpu.VMEM((128, 128), jnp.float32),
                               pltpu.VMEM((128, 128), jnp.float32),
                               pltpu.VMEM_SHARED((_NACC, 128), jnp.float32),
                               pltpu.SemaphoreType.DMA,
                               pltpu.SemaphoreType.DMA,
                               pltpu.SemaphoreType.DMA,
                               pltpu.SemaphoreType.DMA],
            )(_make_sc_scatter_body(nsub))
        _SC_CACHE.append(kernels)
    return _SC_CACHE[0]


def kernel(pos, edge_shifts, cell, emb_table, mlp_w1, mlp_b1, mlp_w2, mlp_b2,
           fc_w1, fc_b1, fc_w2, fc_b2, fc_w3, fc_b3,
           A, batch, edge_src, edge_dst):
    Ai = _node_mlp(A.astype(jnp.int32).reshape(_N, 1), emb_table,
                   mlp_w1, mlp_b1.reshape(1, 64), mlp_w2, mlp_b2.reshape(1, 8))
    ai_ext = jnp.concatenate([Ai, jnp.zeros((1, 8), jnp.float32)], axis=0)
    posp = jnp.concatenate([pos, jnp.zeros((1, 3), jnp.float32)], axis=0)
    px = posp[:, 0]
    py = posp[:, 1]
    pz = posp[:, 2]
    npad = _EPAD - _E
    src2 = jnp.concatenate([edge_src.astype(jnp.int32),
                            jnp.full((npad,), _N, jnp.int32)]).reshape(-1, 128)
    dst2 = jnp.concatenate([edge_dst.astype(jnp.int32),
                            jnp.zeros((npad,), jnp.int32)]).reshape(-1, 128)
    ks = _get_sc_kernels()
    zrows = jnp.zeros((632, 128), jnp.float32)
    wargs = (fc_w1, fc_b1.reshape(1, 64), fc_w2, fc_b2.reshape(1, 64),
             fc_w3, fc_b3.reshape(1, 384))
    r0 = 768
    xs_a, p_a = ks[("g", 3)](px, py, pz, ai_ext, src2[:r0], dst2[:r0])
    feat_a = _edge_dense(p_a, xs_a, *wargs)
    xs_b, p_b = ks[("g", 2)](px, py, pz, ai_ext, src2[r0:], dst2[r0:])
    feat_b = _edge_dense(p_b, xs_b, *wargs)
    acc_a = ks[("s", 3)](feat_a, dst2[:r0], zrows)
    acc_b = ks[("s", 2)](feat_b, dst2[r0:], zrows)
    acc2 = acc_a + acc_b
    acc = jnp.concatenate([acc2[0, :_N, :104], acc2[1, :_N, :104]], axis=1)
    acc3 = acc.reshape(_N, 16, 13)
    blk0 = acc3[:, :, 0]
    blk1 = acc3[:, :, 1:4].reshape(_N, 48)
    blk2 = acc3[:, :, 4:13].reshape(_N, 144)
    return jnp.concatenate(
        [blk0, jnp.zeros((_N, 16), jnp.float32),
         blk1, jnp.zeros((_N, 48), jnp.float32),
         blk2, jnp.zeros((_N, 144), jnp.float32)], axis=1)
